# Initial kernel scaffold; baseline (speedup 1.0000x reference)
import jax
import jax.numpy as jnp
from jax.experimental import pallas as pl


def kernel(x, lap_rows, lap_cols, lap_vals, weight, bias):
    raise NotImplementedError("write your pallas kernel here")


# speedup vs baseline: 4.5761x; 4.5761x over previous
import functools
import jax
import jax.numpy as jnp
from jax import lax
from jax.experimental import pallas as pl
from jax.experimental.pallas import tpu as pltpu
from jax.experimental.pallas import tpu_sc as plsc

V = 10000
E = 320000
B = 4
FIN = 128
FOUT = 128
K = 3

NC = 2
NS = 16
CH = 128
IG = 8
NGRP = 20
EPAD = NS * NGRP * IG * CH
RPT = 624
ZR = 48
NZC = RPT // ZR


def _sc_spmm_pass(src_hbm, dst_hbm, cols_h, rows_h, vals_h, bV, s,
                  accum, cidx, ridx, vbuf, gbuf, zbuf, sems):
    for i in range(NZC):
        pltpu.sync_copy(zbuf, accum.at[pl.ds(s * RPT + i * ZR, ZR)])

    @pl.when(s == NS - 1)
    def _():
        pltpu.sync_copy(zbuf.at[pl.ds(0, 16)], accum.at[pl.ds(NS * RPT, 16)])
    plsc.subcore_barrier()

    @pl.loop(0, NGRP)
    def _group(g):
        rbase = s * (NGRP * IG) + g * IG
        pltpu.sync_copy(cols_h.at[pl.ds(rbase, IG)], cidx)
        pltpu.sync_copy(rows_h.at[pl.ds(rbase, IG)], ridx)
        pltpu.sync_copy(vals_h.at[pl.ds(rbase * CH, IG * CH)],
                        vbuf.at[pl.ds(0, IG * CH)])

        def _offs(u, _):
            sl = pl.ds(u * 16, 16)
            for jj in range(IG):
                cidx[jj, sl] = cidx[jj, sl] + bV
            return _
        lax.fori_loop(0, CH // 16, _offs, None)

        descs = [None, None]
        descs[0] = pltpu.async_copy(src_hbm.at[cidx.at[0]], gbuf.at[0], sems[0])
        for jj in range(IG):
            p = jj & 1
            if jj + 1 < IG:
                descs[1 - p] = pltpu.async_copy(
                    src_hbm.at[cidx.at[jj + 1]], gbuf.at[1 - p], sems[1 - p])
            descs[p].wait()

            def _scale(i, _):
                v16 = vbuf[pl.ds(jj * CH + i, 16)]
                v = jnp.broadcast_to(v16[0], (16,))
                for u in range(FIN // 16):
                    sl = pl.ds(u * 16, 16)
                    gbuf[p, i, sl] = gbuf[p, i, sl] * v
                return _
            lax.fori_loop(0, CH, _scale, None)
            pltpu.sync_copy(gbuf.at[p], accum.at[ridx.at[jj]], add=True)

    plsc.subcore_barrier()
    for i in range(NZC):
        pltpu.sync_copy(accum.at[pl.ds(s * RPT + i * ZR, ZR)],
                        dst_hbm.at[pl.ds(bV + s * RPT + i * ZR, ZR)])

    @pl.when(s == NS - 1)
    def _():
        pltpu.sync_copy(accum.at[pl.ds(NS * RPT, 16)],
                        dst_hbm.at[pl.ds(bV + NS * RPT, 16)])
    plsc.subcore_barrier()


def _sc_body(xflat, cols_h, rows_h, vals_h, s1_out, s2_out,
             accum, cidx, ridx, vbuf, gbuf, zbuf, sem0, sem1):
    c = lax.axis_index("c")
    s = lax.axis_index("s")
    sems = (sem0, sem1)

    def _z(i, _):
        zero = jnp.zeros((16,), jnp.float32)
        for u in range(FIN // 16):
            zbuf[i, pl.ds(u * 16, 16)] = zero
        return _
    lax.fori_loop(0, ZR, _z, None)
    del _z

    @pl.loop(0, 2)
    def _quarter(qi):
        bV = (2 * c + qi) * V
        _sc_spmm_pass(xflat, s1_out, cols_h, rows_h, vals_h, bV, s,
                      accum, cidx, ridx, vbuf, gbuf, zbuf, sems)
        _sc_spmm_pass(s1_out, s2_out, cols_h, rows_h, vals_h, bV, s,
                      accum, cidx, ridx, vbuf, gbuf, zbuf, sems)


def _sc_spmm(xflat, cols2, rows2, vals2):
    mesh = plsc.VectorSubcoreMesh(core_axis_name="c", subcore_axis_name="s")
    f = pl.kernel(
        _sc_body,
        out_type=(
            jax.ShapeDtypeStruct((B * V, FIN), jnp.float32),
            jax.ShapeDtypeStruct((B * V, FIN), jnp.float32),
        ),
        mesh=mesh,
        scratch_types=dict(
            accum=pltpu.VMEM_SHARED((V, FIN), jnp.float32),
            cidx=pltpu.VMEM((IG, CH), jnp.int32),
            ridx=pltpu.VMEM((IG, CH), jnp.int32),
            vbuf=pltpu.VMEM((IG * CH + 16,), jnp.float32),
            gbuf=pltpu.VMEM((2, CH, FIN), jnp.float32),
            zbuf=pltpu.VMEM((ZR, FIN), jnp.float32),
            sem0=pltpu.SemaphoreType.DMA,
            sem1=pltpu.SemaphoreType.DMA,
        ),
    )
    return f(xflat, cols2, rows2, vals2)


def _tc_body(x_ref, s1_ref, s2_ref, a_ref, bias_ref, out_ref):
    x0 = x_ref[...]
    x1 = s1_ref[...]
    x2 = s2_ref[...]
    dn = (((0,), (1,)), ((), ()))
    acc = lax.dot_general(a_ref[0], x0, dn, preferred_element_type=jnp.float32)
    acc += lax.dot_general(a_ref[1], x1, dn, preferred_element_type=jnp.float32)
    acc += lax.dot_general(a_ref[2], x2, dn, preferred_element_type=jnp.float32)
    out_ref[0] = acc + bias_ref[...]


def _tc_matmul(xflat, s1, s2, a_eff, bias2):
    grid = (B,)
    return pl.pallas_call(
        _tc_body,
        grid=grid,
        in_specs=[
            pl.BlockSpec((V, FIN), lambda b: (b, 0)),
            pl.BlockSpec((V, FIN), lambda b: (b, 0)),
            pl.BlockSpec((V, FIN), lambda b: (b, 0)),
            pl.BlockSpec((K, FIN, FOUT), lambda b: (0, 0, 0)),
            pl.BlockSpec((FOUT, 1), lambda b: (0, 0)),
        ],
        out_specs=pl.BlockSpec((1, FOUT, V), lambda b: (b, 0, 0)),
        out_shape=jax.ShapeDtypeStruct((B, FOUT, V), jnp.float32),
    )(xflat, s1, s2, a_eff, bias2)


@jax.jit
def kernel(x, lap_rows, lap_cols, lap_vals, weight, bias):
    xq = jnp.transpose(x, (0, 2, 1)).reshape(B * V, FIN)

    npad = EPAD - E
    pad_idx = (jnp.arange(npad, dtype=jnp.int32) * 16) % V
    cols_p = jnp.concatenate([lap_cols, pad_idx]).reshape(EPAD // CH, CH)
    rows_p = jnp.concatenate([lap_rows, pad_idx]).reshape(EPAD // CH, CH)
    vals_p = jnp.concatenate([lap_vals, jnp.zeros((npad,), jnp.float32)])

    w_flat = weight.reshape(K * FIN, FOUT)
    wp = w_flat.reshape(FIN, K, FOUT).transpose(1, 0, 2)
    a_eff = jnp.stack([wp[0] - wp[2], wp[1], 2.0 * wp[2]])
    bias2 = bias[:, None]

    s1, s2 = _sc_spmm(xq, cols_p, rows_p, vals_p)
    return _tc_matmul(xq, s1, s2, a_eff, bias2)


# speedup vs baseline: 5.6766x; 1.2405x over previous
import functools
import jax
import jax.numpy as jnp
from jax import lax
from jax.experimental import pallas as pl
from jax.experimental.pallas import tpu as pltpu
from jax.experimental.pallas import tpu_sc as plsc

V = 10000
E = 320000
B = 4
FIN = 128
FOUT = 128
K = 3

NC = 2
NS = 16
CH = 128
IG = 8
NGRP = 20
EPAD = NS * NGRP * IG * CH
RPT = 624
ZR = 48
NZC = RPT // ZR


def _sc_spmm_pass(src_hbm, dst_hbm, cols_h, rows_h, vals_h, bV, s,
                  accum, cidx, ridx, vbuf, gbuf, zbuf, sems):
    for i in range(NZC):
        pltpu.sync_copy(zbuf, accum.at[pl.ds(s * RPT + i * ZR, ZR)])

    @pl.when(s == NS - 1)
    def _():
        pltpu.sync_copy(zbuf.at[pl.ds(0, 16)], accum.at[pl.ds(NS * RPT, 16)])
    plsc.subcore_barrier()

    @pl.loop(0, NGRP)
    def _group(g):
        rbase = s * (NGRP * IG) + g * IG
        pltpu.sync_copy(cols_h.at[pl.ds(rbase, IG)], cidx)
        pltpu.sync_copy(rows_h.at[pl.ds(rbase, IG)], ridx)
        pltpu.sync_copy(vals_h.at[pl.ds(rbase * CH, IG * CH)],
                        vbuf.at[pl.ds(0, IG * CH)])

        def _offs(u, _):
            sl = pl.ds(u * 16, 16)
            for jj in range(IG):
                cidx[jj, sl] = cidx[jj, sl] + bV
            return _
        lax.fori_loop(0, CH // 16, _offs, None)

        gd = [None, None]
        sd = [None, None]
        gd[0] = pltpu.async_copy(src_hbm.at[cidx.at[0]], gbuf.at[0], sems[0])
        for jj in range(IG):
            p = jj & 1
            if jj + 1 < IG:
                if jj >= 1:
                    sd[1 - p].wait()
                gd[1 - p] = pltpu.async_copy(
                    src_hbm.at[cidx.at[jj + 1]], gbuf.at[1 - p], sems[1 - p])
            gd[p].wait()

            @plsc.parallel_loop(0, CH, unroll=4)
            def _scale(i):
                v16 = vbuf[pl.ds(jj * CH + i, 16)]
                v = v16[0]
                for u in range(FIN // 16):
                    sl = pl.ds(u * 16, 16)
                    gbuf[p, i, sl] = gbuf[p, i, sl] * v

            sd[p] = pltpu.make_async_copy(
                gbuf.at[p], accum.at[ridx.at[jj]], sems[2 + p])
            sd[p].start(add=True)
        sd[0].wait()
        sd[1].wait()

    plsc.subcore_barrier()
    for i in range(NZC):
        pltpu.sync_copy(accum.at[pl.ds(s * RPT + i * ZR, ZR)],
                        dst_hbm.at[pl.ds(bV + s * RPT + i * ZR, ZR)])

    @pl.when(s == NS - 1)
    def _():
        pltpu.sync_copy(accum.at[pl.ds(NS * RPT, 16)],
                        dst_hbm.at[pl.ds(bV + NS * RPT, 16)])
    plsc.subcore_barrier()


def _sc_body(xflat, cols_h, rows_h, vals_h, s1_out, s2_out,
             accum, cidx, ridx, vbuf, gbuf, zbuf, sem0, sem1, sem2, sem3):
    c = lax.axis_index("c")
    s = lax.axis_index("s")
    sems = (sem0, sem1, sem2, sem3)

    def _z(i, _):
        zero = jnp.zeros((16,), jnp.float32)
        for u in range(FIN // 16):
            zbuf[i, pl.ds(u * 16, 16)] = zero
        return _
    lax.fori_loop(0, ZR, _z, None)
    del _z

    @pl.loop(0, 2)
    def _quarter(qi):
        bV = (2 * c + qi) * V
        _sc_spmm_pass(xflat, s1_out, cols_h, rows_h, vals_h, bV, s,
                      accum, cidx, ridx, vbuf, gbuf, zbuf, sems)
        _sc_spmm_pass(s1_out, s2_out, cols_h, rows_h, vals_h, bV, s,
                      accum, cidx, ridx, vbuf, gbuf, zbuf, sems)


def _sc_spmm(xflat, cols2, rows2, vals2):
    mesh = plsc.VectorSubcoreMesh(core_axis_name="c", subcore_axis_name="s")
    f = pl.kernel(
        _sc_body,
        out_type=(
            jax.ShapeDtypeStruct((B * V, FIN), jnp.float32),
            jax.ShapeDtypeStruct((B * V, FIN), jnp.float32),
        ),
        mesh=mesh,
        scratch_types=dict(
            accum=pltpu.VMEM_SHARED((V, FIN), jnp.float32),
            cidx=pltpu.VMEM((IG, CH), jnp.int32),
            ridx=pltpu.VMEM((IG, CH), jnp.int32),
            vbuf=pltpu.VMEM((IG * CH + 16,), jnp.float32),
            gbuf=pltpu.VMEM((2, CH, FIN), jnp.float32),
            zbuf=pltpu.VMEM((ZR, FIN), jnp.float32),
            sem0=pltpu.SemaphoreType.DMA,
            sem1=pltpu.SemaphoreType.DMA,
            sem2=pltpu.SemaphoreType.DMA,
            sem3=pltpu.SemaphoreType.DMA,
        ),
    )
    return f(xflat, cols2, rows2, vals2)


def _tc_body(x_ref, s1_ref, s2_ref, a_ref, bias_ref, out_ref):
    x0 = x_ref[...]
    x1 = s1_ref[...]
    x2 = s2_ref[...]
    dn = (((0,), (1,)), ((), ()))
    acc = lax.dot_general(a_ref[0], x0, dn, preferred_element_type=jnp.float32)
    acc += lax.dot_general(a_ref[1], x1, dn, preferred_element_type=jnp.float32)
    acc += lax.dot_general(a_ref[2], x2, dn, preferred_element_type=jnp.float32)
    out_ref[0] = acc + bias_ref[...]


def _tc_matmul(xflat, s1, s2, a_eff, bias2):
    grid = (B,)
    return pl.pallas_call(
        _tc_body,
        grid=grid,
        in_specs=[
            pl.BlockSpec((V, FIN), lambda b: (b, 0)),
            pl.BlockSpec((V, FIN), lambda b: (b, 0)),
            pl.BlockSpec((V, FIN), lambda b: (b, 0)),
            pl.BlockSpec((K, FIN, FOUT), lambda b: (0, 0, 0)),
            pl.BlockSpec((FOUT, 1), lambda b: (0, 0)),
        ],
        out_specs=pl.BlockSpec((1, FOUT, V), lambda b: (b, 0, 0)),
        out_shape=jax.ShapeDtypeStruct((B, FOUT, V), jnp.float32),
    )(xflat, s1, s2, a_eff, bias2)


@jax.jit
def kernel(x, lap_rows, lap_cols, lap_vals, weight, bias):
    xq = jnp.transpose(x, (0, 2, 1)).reshape(B * V, FIN)

    npad = EPAD - E
    pad_idx = (jnp.arange(npad, dtype=jnp.int32) * 16) % V
    cols_p = jnp.concatenate([lap_cols, pad_idx]).reshape(EPAD // CH, CH)
    rows_p = jnp.concatenate([lap_rows, pad_idx]).reshape(EPAD // CH, CH)
    vals_p = jnp.concatenate([lap_vals, jnp.zeros((npad,), jnp.float32)])

    w_flat = weight.reshape(K * FIN, FOUT)
    wp = w_flat.reshape(FIN, K, FOUT).transpose(1, 0, 2)
    a_eff = jnp.stack([wp[0] - wp[2], wp[1], 2.0 * wp[2]])
    bias2 = bias[:, None]

    s1, s2 = _sc_spmm(xq, cols_p, rows_p, vals_p)
    return _tc_matmul(xq, s1, s2, a_eff, bias2)


# speedup vs baseline: 6.2277x; 1.0971x over previous
import jax
import jax.numpy as jnp
from jax import lax
from jax.experimental import pallas as pl
from jax.experimental.pallas import tpu as pltpu
from jax.experimental.pallas import tpu_sc as plsc

V = 10000
E = 320000
B = 4
FIN = 128
FOUT = 128
K = 3

NC = 2
NS = 16
CH = 128
IG = 8
NGRP = 20
EPAD = NS * NGRP * IG * CH
NGR = EPAD // CH
RPT = 624
ZR = 48
NZC = RPT // ZR


def _sc_spmm_pass(src_hbm, dst_hbm, cols_h, rows_h, vals_h, bV, brow, s,
                  accum, cidx, ridx, vbuf, gbuf, zbuf, sems):
    for i in range(NZC):
        pltpu.sync_copy(zbuf, accum.at[pl.ds(s * RPT + i * ZR, ZR)])

    @pl.when(s == NS - 1)
    def _():
        pltpu.sync_copy(zbuf.at[pl.ds(0, 16)], accum.at[pl.ds(NS * RPT, 16)])
    plsc.subcore_barrier()

    def _idx_load(g, buf, sem):
        rbase = s * (NGRP * IG) + g * IG
        pltpu.async_copy(cols_h.at[pl.ds(brow + rbase, IG)],
                         cidx.at[buf], sem)
        pltpu.async_copy(rows_h.at[pl.ds(rbase, IG)],
                         ridx.at[buf], sem)
        pltpu.async_copy(vals_h.at[pl.ds(rbase * CH, IG * CH)],
                         vbuf.at[pl.ds(buf * IG * CH, IG * CH)], sem)

    def _idx_drain(buf, sem):
        pltpu.make_async_copy(cols_h.at[pl.ds(0, IG)], cidx.at[buf], sem).wait()
        pltpu.make_async_copy(rows_h.at[pl.ds(0, IG)], ridx.at[buf], sem).wait()
        pltpu.make_async_copy(vals_h.at[pl.ds(0, IG * CH)],
                              vbuf.at[pl.ds(buf * IG * CH, IG * CH)], sem).wait()

    def _process_group(buf):
        gd = [None, None]
        sd = [None, None]
        gd[0] = pltpu.async_copy(src_hbm.at[cidx.at[buf, 0]],
                                 gbuf.at[0], sems[0])
        for jj in range(IG):
            p = jj & 1
            if jj + 1 < IG:
                if jj >= 1:
                    sd[1 - p].wait()
                gd[1 - p] = pltpu.async_copy(
                    src_hbm.at[cidx.at[buf, jj + 1]], gbuf.at[1 - p],
                    sems[1 - p])
            gd[p].wait()

            @plsc.parallel_loop(0, CH, unroll=8)
            def _scale(i):
                v16 = vbuf[pl.ds(buf * IG * CH + jj * CH + i, 16)]
                v = v16[0]
                for u in range(FIN // 16):
                    sl = pl.ds(u * 16, 16)
                    gbuf[p, i, sl] = gbuf[p, i, sl] * v

            sd[p] = pltpu.make_async_copy(
                gbuf.at[p], accum.at[ridx.at[buf, jj]], sems[2 + p])
            sd[p].start(add=True)
        sd[0].wait()
        sd[1].wait()

    _idx_load(0, 0, sems[4])

    @pl.loop(0, NGRP // 2)
    def _group_pair(h):
        g0 = h * 2
        _idx_load(g0 + 1, 1, sems[5])
        _idx_drain(0, sems[4])
        _process_group(0)

        @pl.when(h + 1 < NGRP // 2)
        def _():
            _idx_load(g0 + 2, 0, sems[4])
        _idx_drain(1, sems[5])
        _process_group(1)

    plsc.subcore_barrier()
    for i in range(NZC):
        pltpu.sync_copy(accum.at[pl.ds(s * RPT + i * ZR, ZR)],
                        dst_hbm.at[pl.ds(bV + s * RPT + i * ZR, ZR)])

    @pl.when(s == NS - 1)
    def _():
        pltpu.sync_copy(accum.at[pl.ds(NS * RPT, 16)],
                        dst_hbm.at[pl.ds(bV + NS * RPT, 16)])
    plsc.subcore_barrier()


def _sc_body(xflat, cols_h, rows_h, vals_h, s1_out, s2_out,
             accum, cidx, ridx, vbuf, gbuf, zbuf,
             sem0, sem1, sem2, sem3, sem4, sem5):
    c = lax.axis_index("c")
    s = lax.axis_index("s")
    sems = (sem0, sem1, sem2, sem3, sem4, sem5)

    def _z(i, _):
        zero = jnp.zeros((16,), jnp.float32)
        for u in range(FIN // 16):
            zbuf[i, pl.ds(u * 16, 16)] = zero
        return _
    lax.fori_loop(0, ZR, _z, None)

    @pl.loop(0, 2)
    def _quarter(qi):
        b = 2 * c + qi
        bV = b * V
        brow = b * NGR
        _sc_spmm_pass(xflat, s1_out, cols_h, rows_h, vals_h, bV, brow, s,
                      accum, cidx, ridx, vbuf, gbuf, zbuf, sems)
        _sc_spmm_pass(s1_out, s2_out, cols_h, rows_h, vals_h, bV, brow, s,
                      accum, cidx, ridx, vbuf, gbuf, zbuf, sems)


def _sc_spmm(xflat, cols4, rows2, vals2):
    mesh = plsc.VectorSubcoreMesh(core_axis_name="c", subcore_axis_name="s")
    f = pl.kernel(
        _sc_body,
        out_type=(
            jax.ShapeDtypeStruct((B * V, FIN), jnp.float32),
            jax.ShapeDtypeStruct((B * V, FIN), jnp.float32),
        ),
        mesh=mesh,
        scratch_types=dict(
            accum=pltpu.VMEM_SHARED((V, FIN), jnp.float32),
            cidx=pltpu.VMEM((2, IG, CH), jnp.int32),
            ridx=pltpu.VMEM((2, IG, CH), jnp.int32),
            vbuf=pltpu.VMEM((2 * IG * CH + 16,), jnp.float32),
            gbuf=pltpu.VMEM((2, CH, FIN), jnp.float32),
            zbuf=pltpu.VMEM((ZR, FIN), jnp.float32),
            sem0=pltpu.SemaphoreType.DMA,
            sem1=pltpu.SemaphoreType.DMA,
            sem2=pltpu.SemaphoreType.DMA,
            sem3=pltpu.SemaphoreType.DMA,
            sem4=pltpu.SemaphoreType.DMA,
            sem5=pltpu.SemaphoreType.DMA,
        ),
    )
    return f(xflat, cols4, rows2, vals2)


def _tc_body(x_ref, s1_ref, s2_ref, a_ref, bias_ref, out_ref):
    x0 = x_ref[...]
    x1 = s1_ref[...]
    x2 = s2_ref[...]
    dn = (((0,), (1,)), ((), ()))
    acc = lax.dot_general(a_ref[0], x0, dn, preferred_element_type=jnp.float32)
    acc += lax.dot_general(a_ref[1], x1, dn, preferred_element_type=jnp.float32)
    acc += lax.dot_general(a_ref[2], x2, dn, preferred_element_type=jnp.float32)
    out_ref[0] = acc + bias_ref[...]


def _tc_matmul(xflat, s1, s2, a_eff, bias2):
    grid = (B,)
    return pl.pallas_call(
        _tc_body,
        grid=grid,
        in_specs=[
            pl.BlockSpec((V, FIN), lambda b: (b, 0)),
            pl.BlockSpec((V, FIN), lambda b: (b, 0)),
            pl.BlockSpec((V, FIN), lambda b: (b, 0)),
            pl.BlockSpec((K, FIN, FOUT), lambda b: (0, 0, 0)),
            pl.BlockSpec((FOUT, 1), lambda b: (0, 0)),
        ],
        out_specs=pl.BlockSpec((1, FOUT, V), lambda b: (b, 0, 0)),
        out_shape=jax.ShapeDtypeStruct((B, FOUT, V), jnp.float32),
    )(xflat, s1, s2, a_eff, bias2)


@jax.jit
def kernel(x, lap_rows, lap_cols, lap_vals, weight, bias):
    xq = jnp.transpose(x, (0, 2, 1)).reshape(B * V, FIN)

    npad = EPAD - E
    pad_idx = (jnp.arange(npad, dtype=jnp.int32) * 16) % V
    cols_pad = jnp.concatenate([lap_cols, pad_idx])
    cols4 = (cols_pad[None, :] +
             (jnp.arange(B, dtype=jnp.int32) * V)[:, None]).reshape(B * NGR, CH)
    rows_p = jnp.concatenate([lap_rows, pad_idx]).reshape(NGR, CH)
    vals_p = jnp.concatenate([lap_vals, jnp.zeros((npad,), jnp.float32)])

    w_flat = weight.reshape(K * FIN, FOUT)
    wp = w_flat.reshape(FIN, K, FOUT).transpose(1, 0, 2)
    a_eff = jnp.stack([wp[0] - wp[2], wp[1], 2.0 * wp[2]])
    bias2 = bias[:, None]

    s1, s2 = _sc_spmm(xq, cols4, rows_p, vals_p)
    return _tc_matmul(xq, s1, s2, a_eff, bias2)
